# baseline (speedup 1.0000x reference)
import functools

import jax
import jax.numpy as jnp
from jax import lax
from jax.experimental import pallas as pl
from jax.experimental.pallas import tpu as pltpu
from jax.experimental.pallas import tpu_sc as plsc

VOCAB = 256
EMBED_DIM = 100
D2 = 2 * EMBED_DIM

NC = 2
NS = 16
NW = NC * NS

G = 128
JG = 2
CHUNK = G * JG
NBUF = 2


def _emb_kernel(n_chunks_per_w):
    n_iter = n_chunks_per_w // NBUF

    def body(idx_hbm, table_hbm, out_hbm, idx_v, rows_v, idx_sem, gat_sem,
             out_sem):
        wid = lax.axis_index("s") * NC + lax.axis_index("c")
        base = wid * n_chunks_per_w

        def drain_idx(b):
            pltpu.make_async_copy(idx_hbm.at[0], idx_v.at[b],
                                  idx_sem.at[b]).wait()

        def drain_gat(b):
            pltpu.make_async_copy(out_hbm.at[pl.ds(0, CHUNK)], rows_v.at[b],
                                  gat_sem.at[b]).wait()

        def drain_out(b):
            pltpu.make_async_copy(rows_v.at[b], out_hbm.at[pl.ds(0, CHUNK)],
                                  out_sem.at[b]).wait()

        pltpu.sync_copy(idx_hbm.at[base], idx_v.at[0])

        def loop_body(t, carry):
            for b in range(NBUF):
                g = t * NBUF + b
                if b == 0:
                    @pl.when(t > 0)
                    def _():
                        drain_idx(b)
                        drain_out(b)
                else:
                    drain_idx(b)

                    @pl.when(t > 0)
                    def _():
                        drain_out(b)

                for j in range(JG):
                    pltpu.async_copy(
                        table_hbm.at[idx_v.at[b].at[j]],
                        rows_v.at[b].at[pl.ds(j * G, G)],
                        gat_sem.at[b],
                    )
                bn = (b + 1) % NBUF
                if b < NBUF - 1:
                    pltpu.async_copy(idx_hbm.at[base + g + 1], idx_v.at[bn],
                                     idx_sem.at[bn])
                else:
                    @pl.when(t < n_iter - 1)
                    def _():
                        pltpu.async_copy(idx_hbm.at[base + g + 1],
                                         idx_v.at[bn], idx_sem.at[bn])

                drain_gat(b)
                pltpu.async_copy(rows_v.at[b],
                                 out_hbm.at[pl.ds((base + g) * CHUNK, CHUNK)],
                                 out_sem.at[b])
            return carry

        lax.fori_loop(0, n_iter, loop_body, 0)
        for b in range(NBUF):
            drain_out(b)

    return body


def _pairs_body(table_ref, out_ref):
    a = pl.program_id(0)
    left = jnp.broadcast_to(table_ref[pl.ds(a, 1), :], (VOCAB, EMBED_DIM))
    out_ref[:, 0:EMBED_DIM] = left
    out_ref[:, EMBED_DIM:D2] = table_ref[:, :]


def _build_pairs(table):
    return pl.pallas_call(
        _pairs_body,
        grid=(VOCAB,),
        in_specs=[pl.BlockSpec((VOCAB, EMBED_DIM), lambda a: (0, 0))],
        out_specs=pl.BlockSpec((VOCAB, D2), lambda a: (a, 0)),
        out_shape=jax.ShapeDtypeStruct((VOCAB * VOCAB, D2), jnp.float32),
    )(table)


def kernel(x, table):
    B, S = x.shape
    n = B * S
    npairs = n // 2
    assert npairs % (NW * CHUNK * NBUF) == 0
    n_chunks_per_w = npairs // (NW * CHUNK)

    x2 = x.reshape(npairs, 2).astype(jnp.int32)
    idx2 = (x2[:, 0] * VOCAB + x2[:, 1]).reshape(npairs // CHUNK, JG, G)

    table2 = _build_pairs(table)

    mesh = plsc.VectorSubcoreMesh(core_axis_name="c", subcore_axis_name="s")
    run = functools.partial(
        pl.kernel,
        mesh=mesh,
        out_type=jax.ShapeDtypeStruct((npairs, D2), jnp.float32),
        scratch_types=[
            pltpu.VMEM((NBUF, JG, G), jnp.int32),
            pltpu.VMEM((NBUF, CHUNK, D2), jnp.float32),
            pltpu.SemaphoreType.DMA((NBUF,)),
            pltpu.SemaphoreType.DMA((NBUF,)),
            pltpu.SemaphoreType.DMA((NBUF,)),
        ],
        compiler_params=pltpu.CompilerParams(use_tc_tiling_on_sc=False),
    )(_emb_kernel(n_chunks_per_w))

    out = run(idx2, table2)
    return out.reshape(B, S, EMBED_DIM)


# speedup vs baseline: 1.0885x; 1.0885x over previous
import functools

import jax
import jax.numpy as jnp
from jax import lax
from jax.experimental import pallas as pl
from jax.experimental.pallas import tpu as pltpu
from jax.experimental.pallas import tpu_sc as plsc

VOCAB = 256
EMBED_DIM = 100
D2 = 2 * EMBED_DIM

NC = 2
NS = 16
NW = NC * NS

G = 128
JG = 2
CHUNK = G * JG
NBUF = 2


def _emb_kernel(n_chunks_per_w):
    n_iter = n_chunks_per_w // NBUF

    def body(idx_hbm, table_hbm, out_hbm, idx_v, rows_v, idx_sem, gat_sem,
             out_sem):
        wid = lax.axis_index("s") * NC + lax.axis_index("c")
        base = wid * n_chunks_per_w

        def drain_idx(b):
            pltpu.make_async_copy(idx_hbm.at[0], idx_v.at[b],
                                  idx_sem.at[b]).wait()

        def drain_gat(b):
            pltpu.make_async_copy(out_hbm.at[pl.ds(0, CHUNK)], rows_v.at[b],
                                  gat_sem.at[b]).wait()

        def drain_out(b):
            pltpu.make_async_copy(rows_v.at[b], out_hbm.at[pl.ds(0, CHUNK)],
                                  out_sem.at[b]).wait()

        pltpu.sync_copy(idx_hbm.at[base], idx_v.at[0])

        def loop_body(t, carry):
            for b in range(NBUF):
                g = t * NBUF + b
                if b == 0:
                    @pl.when(t > 0)
                    def _():
                        drain_idx(b)
                        drain_out(b)
                else:
                    drain_idx(b)

                    @pl.when(t > 0)
                    def _():
                        drain_out(b)

                for j in range(JG):
                    pltpu.async_copy(
                        table_hbm.at[idx_v.at[b].at[j]],
                        rows_v.at[b].at[pl.ds(j * G, G)],
                        gat_sem.at[b],
                    )
                bn = (b + 1) % NBUF
                if b < NBUF - 1:
                    pltpu.async_copy(idx_hbm.at[base + g + 1], idx_v.at[bn],
                                     idx_sem.at[bn])
                else:
                    @pl.when(t < n_iter - 1)
                    def _():
                        pltpu.async_copy(idx_hbm.at[base + g + 1],
                                         idx_v.at[bn], idx_sem.at[bn])

                drain_gat(b)
                pltpu.async_copy(rows_v.at[b],
                                 out_hbm.at[pl.ds((base + g) * CHUNK, CHUNK)],
                                 out_sem.at[b])
            return carry

        lax.fori_loop(0, n_iter, loop_body, 0)
        for b in range(NBUF):
            drain_out(b)

    return body


def _pairs_body(table_ref, out_ref):
    a = pl.program_id(0)
    left = jnp.broadcast_to(table_ref[pl.ds(a, 1), :], (VOCAB, EMBED_DIM))
    out_ref[:, 0:EMBED_DIM] = left
    out_ref[:, EMBED_DIM:D2] = table_ref[:, :]


def _build_pairs(table):
    return pl.pallas_call(
        _pairs_body,
        grid=(VOCAB,),
        in_specs=[pl.BlockSpec((VOCAB, EMBED_DIM), lambda a: (0, 0))],
        out_specs=pl.BlockSpec((VOCAB, D2), lambda a: (a, 0)),
        out_shape=jax.ShapeDtypeStruct((VOCAB * VOCAB, D2), jnp.float32),
    )(table)


def kernel(x, table):
    B, S = x.shape
    n = B * S
    npairs = n // 2
    assert npairs % (NW * CHUNK * NBUF) == 0
    n_chunks_per_w = npairs // (NW * CHUNK)

    x2f = x.reshape(npairs, 2).astype(jnp.float32)
    idx2 = (x2f @ jnp.array([float(VOCAB), 1.0], jnp.float32)).astype(
        jnp.int32).reshape(npairs // CHUNK, JG, G)

    table2 = _build_pairs(table)

    mesh = plsc.VectorSubcoreMesh(core_axis_name="c", subcore_axis_name="s")
    run = functools.partial(
        pl.kernel,
        mesh=mesh,
        out_type=jax.ShapeDtypeStruct((npairs, D2), jnp.float32),
        scratch_types=[
            pltpu.VMEM((NBUF, JG, G), jnp.int32),
            pltpu.VMEM((NBUF, CHUNK, D2), jnp.float32),
            pltpu.SemaphoreType.DMA((NBUF,)),
            pltpu.SemaphoreType.DMA((NBUF,)),
            pltpu.SemaphoreType.DMA((NBUF,)),
        ],
        compiler_params=pltpu.CompilerParams(use_tc_tiling_on_sc=False),
    )(_emb_kernel(n_chunks_per_w))

    out = run(idx2, table2)
    return out.reshape(B, S, EMBED_DIM)


# speedup vs baseline: 1.1010x; 1.0115x over previous
import functools

import jax
import jax.numpy as jnp
from jax import lax
from jax.experimental import pallas as pl
from jax.experimental.pallas import tpu as pltpu
from jax.experimental.pallas import tpu_sc as plsc

VOCAB = 256
EMBED_DIM = 100
D2 = 2 * EMBED_DIM

NC = 2
NS = 16
NW = NC * NS

G = 128
JG = 2
CHUNK = G * JG
NBUF = 2

BB = 64
A_PER_S = VOCAB // NS
NBLK = A_PER_S * (VOCAB // BB)


def _emb_kernel(n_chunks_per_w):
    n_iter = n_chunks_per_w // NBUF

    def body(idx_hbm, table_hbm, out_hbm, t2_hbm, tab_v, idx_v,
             rows_v, bld_sem, idx_sem, gat_sem, out_sem):
        c = lax.axis_index("c")
        s = lax.axis_index("s")
        wid = s * NC + c
        base = wid * n_chunks_per_w
        t2c = t2_hbm.at[c]

        pltpu.sync_copy(table_hbm, tab_v)

        def bslot(u):
            return rows_v.at[0].at[pl.ds(u * BB, BB)]

        def drain_bld(u):
            pltpu.make_async_copy(bslot(u), t2c.at[pl.ds(0, BB)],
                                  bld_sem.at[u]).wait()

        def build_blk(blk, carry):
            u = blk % 2
            a = s * A_PER_S + blk // (VOCAB // BB)
            bb = (blk % (VOCAB // BB)) * BB

            @pl.when(blk >= 2)
            def _():
                drain_bld(u)

            bref = bslot(u)
            a_regs = [tab_v[a, pl.ds(k * 16, 16)] for k in range(6)]
            a_tail = tab_v[a, pl.ds(84, 16)]

            def build_row(r, carry2):
                for k in range(6):
                    bref[r, pl.ds(k * 16, 16)] = a_regs[k]
                bref[r, pl.ds(84, 16)] = a_tail
                for j in range(6):
                    bref[r, pl.ds(100 + j * 16, 16)] = \
                        tab_v[bb + r, pl.ds(j * 16, 16)]
                bref[r, pl.ds(184, 16)] = tab_v[bb + r, pl.ds(84, 16)]
                return carry2

            lax.fori_loop(0, BB, build_row, 0)
            pltpu.async_copy(bslot(u),
                             t2c.at[pl.ds(a * VOCAB + bb, BB)],
                             bld_sem.at[u])
            return carry

        lax.fori_loop(0, NBLK, build_blk, 0)
        for u in range(2):
            drain_bld(u)
        plsc.subcore_barrier()

        def drain_idx(b):
            pltpu.make_async_copy(idx_hbm.at[0], idx_v.at[b],
                                  idx_sem.at[b]).wait()

        def drain_gat(b):
            pltpu.make_async_copy(out_hbm.at[pl.ds(0, CHUNK)], rows_v.at[b],
                                  gat_sem.at[b]).wait()

        def drain_out(b):
            pltpu.make_async_copy(rows_v.at[b], out_hbm.at[pl.ds(0, CHUNK)],
                                  out_sem.at[b]).wait()

        pltpu.sync_copy(idx_hbm.at[base], idx_v.at[0])

        def loop_body(t, carry):
            for b in range(NBUF):
                g = t * NBUF + b
                if b == 0:
                    @pl.when(t > 0)
                    def _():
                        drain_idx(b)
                        drain_out(b)
                else:
                    drain_idx(b)

                    @pl.when(t > 0)
                    def _():
                        drain_out(b)

                for j in range(JG):
                    pltpu.async_copy(
                        t2c.at[idx_v.at[b].at[j]],
                        rows_v.at[b].at[pl.ds(j * G, G)],
                        gat_sem.at[b],
                    )
                bn = (b + 1) % NBUF
                if b < NBUF - 1:
                    pltpu.async_copy(idx_hbm.at[base + g + 1], idx_v.at[bn],
                                     idx_sem.at[bn])
                else:
                    @pl.when(t < n_iter - 1)
                    def _():
                        pltpu.async_copy(idx_hbm.at[base + g + 1],
                                         idx_v.at[bn], idx_sem.at[bn])

                drain_gat(b)
                pltpu.async_copy(rows_v.at[b],
                                 out_hbm.at[pl.ds((base + g) * CHUNK, CHUNK)],
                                 out_sem.at[b])
            return carry

        lax.fori_loop(0, n_iter, loop_body, 0)
        for b in range(NBUF):
            drain_out(b)

    return body


def kernel(x, table):
    B, S = x.shape
    n = B * S
    npairs = n // 2
    assert npairs % (NW * CHUNK * NBUF) == 0
    n_chunks_per_w = npairs // (NW * CHUNK)

    x2f = x.reshape(npairs, 2).astype(jnp.float32)
    idx2 = (x2f @ jnp.array([float(VOCAB), 1.0], jnp.float32)).astype(
        jnp.int32).reshape(npairs // CHUNK, JG, G)

    mesh = plsc.VectorSubcoreMesh(core_axis_name="c", subcore_axis_name="s")
    run = functools.partial(
        pl.kernel,
        mesh=mesh,
        out_type=(
            jax.ShapeDtypeStruct((npairs, D2), jnp.float32),
            jax.ShapeDtypeStruct((NC, VOCAB * VOCAB, D2), jnp.float32),
        ),
        scratch_types=[
            pltpu.VMEM((VOCAB, EMBED_DIM), jnp.float32),
            pltpu.VMEM((NBUF, JG, G), jnp.int32),
            pltpu.VMEM((NBUF, CHUNK, D2), jnp.float32),
            pltpu.SemaphoreType.DMA((2,)),
            pltpu.SemaphoreType.DMA((NBUF,)),
            pltpu.SemaphoreType.DMA((NBUF,)),
            pltpu.SemaphoreType.DMA((NBUF,)),
        ],
        compiler_params=pltpu.CompilerParams(use_tc_tiling_on_sc=False),
    )(_emb_kernel(n_chunks_per_w))

    out, _ = run(idx2, table.astype(jnp.float32))
    return out.reshape(B, S, EMBED_DIM)


# speedup vs baseline: 1.1943x; 1.0847x over previous
import functools

import jax
import jax.numpy as jnp
from jax import lax
from jax.experimental import pallas as pl
from jax.experimental.pallas import tpu as pltpu
from jax.experimental.pallas import tpu_sc as plsc

VOCAB = 256
EMBED_DIM = 100
D2 = 2 * EMBED_DIM

NC = 2
NS = 16
NW = NC * NS

G = 128
JG = 2
CHUNK = G * JG
NBUF = 2

BB = 64
A_PER_S = VOCAB // NS
NBLK = A_PER_S * (VOCAB // BB)


def _emb_kernel(n_chunks_per_w):
    n_iter = n_chunks_per_w // NBUF

    def body(x_hbm, table_hbm, out_hbm, t2_hbm, tab_v, xraw_v, idx_v,
             rows_v, bld_sem, xr_sem, gat_sem, out_sem):
        c = lax.axis_index("c")
        s = lax.axis_index("s")
        wid = s * NC + c
        base = wid * n_chunks_per_w
        t2c = t2_hbm.at[c]

        pltpu.sync_copy(table_hbm, tab_v)

        def bslot(u):
            return rows_v.at[0].at[pl.ds(u * BB, BB)]

        def drain_bld(u):
            pltpu.make_async_copy(bslot(u), t2c.at[pl.ds(0, BB)],
                                  bld_sem.at[u]).wait()

        def build_blk(blk, carry):
            u = blk % 2
            a = s * A_PER_S + blk // (VOCAB // BB)
            bb = (blk % (VOCAB // BB)) * BB

            @pl.when(blk >= 2)
            def _():
                drain_bld(u)

            bref = bslot(u)
            a_regs = [tab_v[a, pl.ds(k * 16, 16)] for k in range(6)]
            a_tail = tab_v[a, pl.ds(84, 16)]

            def build_row(r, carry2):
                for k in range(6):
                    bref[r, pl.ds(k * 16, 16)] = a_regs[k]
                bref[r, pl.ds(84, 16)] = a_tail
                for j in range(6):
                    bref[r, pl.ds(100 + j * 16, 16)] = \
                        tab_v[bb + r, pl.ds(j * 16, 16)]
                bref[r, pl.ds(184, 16)] = tab_v[bb + r, pl.ds(84, 16)]
                return carry2

            lax.fori_loop(0, BB, build_row, 0)
            pltpu.async_copy(bslot(u),
                             t2c.at[pl.ds(a * VOCAB + bb, BB)],
                             bld_sem.at[u])
            return carry

        lax.fori_loop(0, NBLK, build_blk, 0)
        for u in range(2):
            drain_bld(u)
        plsc.subcore_barrier()

        even = lax.iota(jnp.int32, 16) * 2

        def drain_xr(b):
            pltpu.make_async_copy(x_hbm.at[pl.ds(0, 2 * CHUNK)],
                                  xraw_v.at[b], xr_sem.at[b]).wait()

        def drain_gat(b):
            pltpu.make_async_copy(out_hbm.at[pl.ds(0, CHUNK)], rows_v.at[b],
                                  gat_sem.at[b]).wait()

        def drain_out(b):
            pltpu.make_async_copy(rows_v.at[b], out_hbm.at[pl.ds(0, CHUNK)],
                                  out_sem.at[b]).wait()

        def make_indices(b):
            xr = xraw_v.at[b]
            for j in range(JG):
                for k in range(G // 16):
                    pos = even + (j * G + k * 16) * 2
                    va = plsc.load_gather(xr, [pos])
                    vb = plsc.load_gather(xr, [pos + 1])
                    idx_v.at[b][j, pl.ds(k * 16, 16)] = va * VOCAB + vb

        pltpu.sync_copy(x_hbm.at[pl.ds(base * 2 * CHUNK, 2 * CHUNK)],
                        xraw_v.at[0])

        def loop_body(t, carry):
            for b in range(NBUF):
                g = t * NBUF + b
                if b == 0:
                    @pl.when(t > 0)
                    def _():
                        drain_xr(b)
                        drain_out(b)
                else:
                    drain_xr(b)

                    @pl.when(t > 0)
                    def _():
                        drain_out(b)

                make_indices(b)
                for j in range(JG):
                    pltpu.async_copy(
                        t2c.at[idx_v.at[b].at[j]],
                        rows_v.at[b].at[pl.ds(j * G, G)],
                        gat_sem.at[b],
                    )
                bn = (b + 1) % NBUF

                def prefetch():
                    pltpu.async_copy(
                        x_hbm.at[pl.ds((base + g + 1) * 2 * CHUNK,
                                       2 * CHUNK)],
                        xraw_v.at[bn], xr_sem.at[bn])

                if b < NBUF - 1:
                    prefetch()
                else:
                    @pl.when(t < n_iter - 1)
                    def _():
                        prefetch()

                drain_gat(b)
                pltpu.async_copy(rows_v.at[b],
                                 out_hbm.at[pl.ds((base + g) * CHUNK, CHUNK)],
                                 out_sem.at[b])
            return carry

        lax.fori_loop(0, n_iter, loop_body, 0)
        for b in range(NBUF):
            drain_out(b)

    return body


def kernel(x, table):
    B, S = x.shape
    n = B * S
    npairs = n // 2
    assert npairs % (NW * CHUNK * NBUF) == 0
    n_chunks_per_w = npairs // (NW * CHUNK)

    xf = x.reshape(n).astype(jnp.int32)

    mesh = plsc.VectorSubcoreMesh(core_axis_name="c", subcore_axis_name="s")
    run = functools.partial(
        pl.kernel,
        mesh=mesh,
        out_type=(
            jax.ShapeDtypeStruct((npairs, D2), jnp.float32),
            jax.ShapeDtypeStruct((NC, VOCAB * VOCAB, D2), jnp.float32),
        ),
        scratch_types=[
            pltpu.VMEM((VOCAB, EMBED_DIM), jnp.float32),
            pltpu.VMEM((NBUF, 2 * CHUNK), jnp.int32),
            pltpu.VMEM((NBUF, JG, G), jnp.int32),
            pltpu.VMEM((NBUF, CHUNK, D2), jnp.float32),
            pltpu.SemaphoreType.DMA((2,)),
            pltpu.SemaphoreType.DMA((NBUF,)),
            pltpu.SemaphoreType.DMA((NBUF,)),
            pltpu.SemaphoreType.DMA((NBUF,)),
        ],
        compiler_params=pltpu.CompilerParams(use_tc_tiling_on_sc=False,
                                             needs_layout_passes=False),
    )(_emb_kernel(n_chunks_per_w))

    out, _ = run(xf, table.astype(jnp.float32))
    return out.reshape(B, S, EMBED_DIM)


# speedup vs baseline: 1.3637x; 1.1418x over previous
import functools

import jax
import jax.numpy as jnp
from jax import lax
from jax.experimental import pallas as pl
from jax.experimental.pallas import tpu as pltpu
from jax.experimental.pallas import tpu_sc as plsc

VOCAB = 256
EMBED_DIM = 100
DPAD = 128

NC = 2
NS = 16
NW = NC * NS

G = 128
JG = 2
CHUNK = G * JG
NBUF = 2


def _emb_kernel(n_chunks_per_w):
    n_iter = n_chunks_per_w // NBUF

    def body(idx_hbm, table_hbm, out_hbm, idx_v, rows_v, idx_sem, gat_sem,
             out_sem):
        wid = lax.axis_index("s") * NC + lax.axis_index("c")
        base = wid * n_chunks_per_w

        def drain_idx(b):
            pltpu.make_async_copy(idx_hbm.at[0], idx_v.at[b],
                                  idx_sem.at[b]).wait()

        def drain_gat(b):
            pltpu.make_async_copy(out_hbm.at[pl.ds(0, CHUNK)], rows_v.at[b],
                                  gat_sem.at[b]).wait()

        def drain_out(b):
            pltpu.make_async_copy(rows_v.at[b], out_hbm.at[pl.ds(0, CHUNK)],
                                  out_sem.at[b]).wait()

        pltpu.sync_copy(idx_hbm.at[base], idx_v.at[0])

        def loop_body(t, carry):
            for b in range(NBUF):
                g = t * NBUF + b
                if b == 0:
                    @pl.when(t > 0)
                    def _():
                        drain_idx(b)
                        drain_out(b)
                else:
                    drain_idx(b)

                    @pl.when(t > 0)
                    def _():
                        drain_out(b)

                for j in range(JG):
                    pltpu.async_copy(
                        table_hbm.at[idx_v.at[b].at[j]],
                        rows_v.at[b].at[pl.ds(j * G, G)],
                        gat_sem.at[b],
                    )
                bn = (b + 1) % NBUF
                if b < NBUF - 1:
                    pltpu.async_copy(idx_hbm.at[base + g + 1], idx_v.at[bn],
                                     idx_sem.at[bn])
                else:
                    @pl.when(t < n_iter - 1)
                    def _():
                        pltpu.async_copy(idx_hbm.at[base + g + 1],
                                         idx_v.at[bn], idx_sem.at[bn])

                drain_gat(b)
                pltpu.async_copy(rows_v.at[b],
                                 out_hbm.at[pl.ds((base + g) * CHUNK, CHUNK)],
                                 out_sem.at[b])
            return carry

        lax.fori_loop(0, n_iter, loop_body, 0)
        for b in range(NBUF):
            drain_out(b)

    return body


def kernel(x, table):
    B, S = x.shape
    n = B * S
    assert n % (NW * CHUNK * NBUF) == 0
    n_chunks_per_w = n // (NW * CHUNK)

    idx = x.reshape(n // CHUNK, JG, G).astype(jnp.int32)
    table_p = jnp.pad(table, ((0, 0), (0, DPAD - EMBED_DIM)))

    mesh = plsc.VectorSubcoreMesh(core_axis_name="c", subcore_axis_name="s")
    run = functools.partial(
        pl.kernel,
        mesh=mesh,
        out_type=jax.ShapeDtypeStruct((n, DPAD), jnp.float32),
        scratch_types=[
            pltpu.VMEM((NBUF, JG, G), jnp.int32),
            pltpu.VMEM((NBUF, CHUNK, DPAD), jnp.float32),
            pltpu.SemaphoreType.DMA((NBUF,)),
            pltpu.SemaphoreType.DMA((NBUF,)),
            pltpu.SemaphoreType.DMA((NBUF,)),
        ],
        compiler_params=pltpu.CompilerParams(use_tc_tiling_on_sc=False),
    )(_emb_kernel(n_chunks_per_w))

    out = run(idx, table_p)
    return out.reshape(B, S, DPAD)[:, :, :EMBED_DIM]


# speedup vs baseline: 2.6027x; 1.9085x over previous
import functools

import jax
import jax.numpy as jnp
from jax import lax
from jax.experimental import pallas as pl
from jax.experimental.pallas import tpu as pltpu
from jax.experimental.pallas import tpu_sc as plsc

VOCAB = 256
EMBED_DIM = 100
DPAD = 128

NC = 2
NS = 16
NW = NC * NS

G = 128
JG = 2
CHUNK = G * JG
NBUF = 2


def _emb_kernel(n_chunks_per_w):
    n_iter = n_chunks_per_w // NBUF

    def body(idx_hbm, table_hbm, out_hbm, idx_v, rows_v, idx_sem, gat_sem,
             out_sem):
        wid = lax.axis_index("s") * NC + lax.axis_index("c")
        base = wid * n_chunks_per_w

        def drain_idx(b):
            pltpu.make_async_copy(idx_hbm.at[0], idx_v.at[b],
                                  idx_sem.at[b]).wait()

        def drain_gat(b):
            pltpu.make_async_copy(out_hbm.at[pl.ds(0, CHUNK)], rows_v.at[b],
                                  gat_sem.at[b]).wait()

        def drain_out(b):
            pltpu.make_async_copy(rows_v.at[b], out_hbm.at[pl.ds(0, CHUNK)],
                                  out_sem.at[b]).wait()

        pltpu.sync_copy(idx_hbm.at[base], idx_v.at[0])

        bias = wid * VOCAB

        def bias_idx(b):
            for j in range(JG):
                for k in range(G // 16):
                    idx_v.at[b][j, pl.ds(k * 16, 16)] = \
                        idx_v.at[b][j, pl.ds(k * 16, 16)] + bias

        def loop_body(t, carry):
            for b in range(NBUF):
                g = t * NBUF + b
                if b == 0:
                    @pl.when(t > 0)
                    def _():
                        drain_idx(b)
                        drain_out(b)
                else:
                    drain_idx(b)

                    @pl.when(t > 0)
                    def _():
                        drain_out(b)

                bias_idx(b)
                for j in range(JG):
                    pltpu.async_copy(
                        table_hbm.at[idx_v.at[b].at[j]],
                        rows_v.at[b].at[pl.ds(j * G, G)],
                        gat_sem.at[b],
                    )
                bn = (b + 1) % NBUF
                if b < NBUF - 1:
                    pltpu.async_copy(idx_hbm.at[base + g + 1], idx_v.at[bn],
                                     idx_sem.at[bn])
                else:
                    @pl.when(t < n_iter - 1)
                    def _():
                        pltpu.async_copy(idx_hbm.at[base + g + 1],
                                         idx_v.at[bn], idx_sem.at[bn])

                drain_gat(b)
                pltpu.async_copy(rows_v.at[b],
                                 out_hbm.at[pl.ds((base + g) * CHUNK, CHUNK)],
                                 out_sem.at[b])
            return carry

        lax.fori_loop(0, n_iter, loop_body, 0)
        for b in range(NBUF):
            drain_out(b)

    return body


def kernel(x, table):
    B, S = x.shape
    n = B * S
    assert n % (NW * CHUNK * NBUF) == 0
    n_chunks_per_w = n // (NW * CHUNK)

    idx = x.reshape(n // CHUNK, JG, G).astype(jnp.int32)
    table_p = jnp.tile(jnp.pad(table, ((0, 0), (0, DPAD - EMBED_DIM))),
                       (NW, 1))

    mesh = plsc.VectorSubcoreMesh(core_axis_name="c", subcore_axis_name="s")
    run = functools.partial(
        pl.kernel,
        mesh=mesh,
        out_type=jax.ShapeDtypeStruct((n, DPAD), jnp.float32),
        scratch_types=[
            pltpu.VMEM((NBUF, JG, G), jnp.int32),
            pltpu.VMEM((NBUF, CHUNK, DPAD), jnp.float32),
            pltpu.SemaphoreType.DMA((NBUF,)),
            pltpu.SemaphoreType.DMA((NBUF,)),
            pltpu.SemaphoreType.DMA((NBUF,)),
        ],
        compiler_params=pltpu.CompilerParams(use_tc_tiling_on_sc=False),
    )(_emb_kernel(n_chunks_per_w))

    out = run(idx, table_p)
    return out.reshape(B, S, DPAD)[:, :, :EMBED_DIM]


# speedup vs baseline: 2.6464x; 1.0168x over previous
import functools

import jax
import jax.numpy as jnp
from jax import lax
from jax.experimental import pallas as pl
from jax.experimental.pallas import tpu as pltpu
from jax.experimental.pallas import tpu_sc as plsc

VOCAB = 256
EMBED_DIM = 100
DPAD = 128

NC = 2
NS = 16
NW = NC * NS

G = 128
CHUNK = G
NBUF = 4


def _emb_kernel(n_chunks_per_w):
    n_iter = n_chunks_per_w // NBUF

    def body(idx_hbm, table_hbm, out_hbm, idx_v, rows_v, idx_sem, gat_sem,
             out_sem):
        wid = lax.axis_index("s") * NC + lax.axis_index("c")
        base = wid * n_chunks_per_w
        bias = wid * VOCAB

        def drain_idx(b):
            pltpu.make_async_copy(idx_hbm.at[0], idx_v.at[b],
                                  idx_sem.at[b]).wait()

        def drain_gat(b):
            pltpu.make_async_copy(out_hbm.at[pl.ds(0, CHUNK)], rows_v.at[b],
                                  gat_sem.at[b]).wait()

        def drain_out(b):
            pltpu.make_async_copy(rows_v.at[b], out_hbm.at[pl.ds(0, CHUNK)],
                                  out_sem.at[b]).wait()

        def bias_idx(b):
            for k in range(G // 16):
                idx_v.at[b][0, pl.ds(k * 16, 16)] = \
                    idx_v.at[b][0, pl.ds(k * 16, 16)] + bias

        def fetch_idx(g, b):
            pltpu.async_copy(idx_hbm.at[base + g], idx_v.at[b],
                             idx_sem.at[b])

        def fire_gather(b):
            pltpu.async_copy(table_hbm.at[idx_v.at[b].at[0]], rows_v.at[b],
                             gat_sem.at[b])

        def fire_writeback(g, b):
            pltpu.async_copy(rows_v.at[b],
                             out_hbm.at[pl.ds((base + g) * CHUNK, CHUNK)],
                             out_sem.at[b])

        for b in range(NBUF):
            fetch_idx(b, b)

        def loop_body(t, carry):
            for b in range(NBUF):
                g = t * NBUF + b
                drain_idx(b)

                @pl.when(t > 0)
                def _():
                    drain_out(b)

                bias_idx(b)
                fire_gather(b)

                b2 = (b - 2) % NBUF

                def complete(gp):
                    drain_gat(b2)
                    fire_writeback(gp, b2)

                if b >= 2:
                    complete(g - 2)

                    @pl.when(t < n_iter - 1)
                    def _():
                        fetch_idx(g + 2, b2)
                else:
                    @pl.when(t > 0)
                    def _():
                        complete(g - 2)
                        fetch_idx(g + 2, b2)
            return carry

        lax.fori_loop(0, n_iter, loop_body, 0)
        last = n_chunks_per_w
        for gp in (last - 2, last - 1):
            b2 = gp % NBUF
            drain_gat(b2)
            fire_writeback(gp, b2)
        for b in range(NBUF):
            drain_out(b)

    return body


def kernel(x, table):
    B, S = x.shape
    n = B * S
    assert n % (NW * CHUNK * NBUF) == 0
    n_chunks_per_w = n // (NW * CHUNK)

    idx = x.reshape(n // CHUNK, 1, G).astype(jnp.int32)
    table_p = jnp.tile(jnp.pad(table, ((0, 0), (0, DPAD - EMBED_DIM))),
                       (NW, 1))

    mesh = plsc.VectorSubcoreMesh(core_axis_name="c", subcore_axis_name="s")
    run = functools.partial(
        pl.kernel,
        mesh=mesh,
        out_type=jax.ShapeDtypeStruct((n, DPAD), jnp.float32),
        scratch_types=[
            pltpu.VMEM((NBUF, 1, G), jnp.int32),
            pltpu.VMEM((NBUF, CHUNK, DPAD), jnp.float32),
            pltpu.SemaphoreType.DMA((NBUF,)),
            pltpu.SemaphoreType.DMA((NBUF,)),
            pltpu.SemaphoreType.DMA((NBUF,)),
        ],
        compiler_params=pltpu.CompilerParams(use_tc_tiling_on_sc=False),
    )(_emb_kernel(n_chunks_per_w))

    out = run(idx, table_p)
    return out.reshape(B, S, DPAD)[:, :, :EMBED_DIM]


# speedup vs baseline: 2.7082x; 1.0234x over previous
import functools

import jax
import jax.numpy as jnp
from jax import lax
from jax.experimental import pallas as pl
from jax.experimental.pallas import tpu as pltpu
from jax.experimental.pallas import tpu_sc as plsc

VOCAB = 256
EMBED_DIM = 100
DPAD = 128

NC = 2
NS = 16
NW = NC * NS

G = 128
CHUNK = G
NBUF = 4


def _emb_kernel(n_chunks_per_w):
    n_iter = n_chunks_per_w // NBUF

    def body(idx_hbm, table_hbm, out_hbm, idx_v, rows_v, idx_sem, gat_sem,
             out_sem):
        wid = lax.axis_index("s") * NC + lax.axis_index("c")
        base = wid * n_chunks_per_w
        bias = wid * VOCAB

        def drain_idx(b):
            pltpu.make_async_copy(idx_hbm.at[0], idx_v.at[b],
                                  idx_sem.at[b]).wait()

        def drain_gat(b):
            pltpu.make_async_copy(out_hbm.at[pl.ds(0, CHUNK)], rows_v.at[b],
                                  gat_sem.at[b]).wait()

        def drain_out(b):
            pltpu.make_async_copy(rows_v.at[b], out_hbm.at[pl.ds(0, CHUNK)],
                                  out_sem.at[b]).wait()

        def bias_idx(b):
            slot_bias = bias * NBUF + b * VOCAB
            for k in range(G // 16):
                idx_v.at[b][0, pl.ds(k * 16, 16)] = \
                    idx_v.at[b][0, pl.ds(k * 16, 16)] + slot_bias

        def fetch_idx(g, b):
            pltpu.async_copy(idx_hbm.at[base + g], idx_v.at[b],
                             idx_sem.at[b])

        def fire_gather(b):
            pltpu.async_copy(table_hbm.at[idx_v.at[b].at[0]], rows_v.at[b],
                             gat_sem.at[b])

        def fire_writeback(g, b):
            pltpu.async_copy(rows_v.at[b],
                             out_hbm.at[pl.ds((base + g) * CHUNK, CHUNK)],
                             out_sem.at[b])

        for b in range(NBUF):
            fetch_idx(b, b)

        def loop_body(t, carry):
            for b in range(NBUF):
                g = t * NBUF + b
                drain_idx(b)

                @pl.when(t > 0)
                def _():
                    drain_out(b)

                bias_idx(b)
                fire_gather(b)

                b2 = (b - 2) % NBUF

                def complete(gp):
                    drain_gat(b2)
                    fire_writeback(gp, b2)

                if b >= 2:
                    complete(g - 2)

                    @pl.when(t < n_iter - 1)
                    def _():
                        fetch_idx(g + 2, b2)
                else:
                    @pl.when(t > 0)
                    def _():
                        complete(g - 2)
                        fetch_idx(g + 2, b2)
            return carry

        lax.fori_loop(0, n_iter, loop_body, 0)
        last = n_chunks_per_w
        for gp in (last - 2, last - 1):
            b2 = gp % NBUF
            drain_gat(b2)
            fire_writeback(gp, b2)
        for b in range(NBUF):
            drain_out(b)

    return body


def kernel(x, table):
    B, S = x.shape
    n = B * S
    assert n % (NW * CHUNK * NBUF) == 0
    n_chunks_per_w = n // (NW * CHUNK)

    idx = x.reshape(n // CHUNK, 1, G).astype(jnp.int32)
    table_p = jnp.tile(jnp.pad(table, ((0, 0), (0, DPAD - EMBED_DIM))),
                       (NW * NBUF, 1))

    mesh = plsc.VectorSubcoreMesh(core_axis_name="c", subcore_axis_name="s")
    run = functools.partial(
        pl.kernel,
        mesh=mesh,
        out_type=jax.ShapeDtypeStruct((n, DPAD), jnp.float32),
        scratch_types=[
            pltpu.VMEM((NBUF, 1, G), jnp.int32),
            pltpu.VMEM((NBUF, CHUNK, DPAD), jnp.float32),
            pltpu.SemaphoreType.DMA((NBUF,)),
            pltpu.SemaphoreType.DMA((NBUF,)),
            pltpu.SemaphoreType.DMA((NBUF,)),
        ],
        compiler_params=pltpu.CompilerParams(use_tc_tiling_on_sc=False),
    )(_emb_kernel(n_chunks_per_w))

    out = run(idx, table_p)
    return out.reshape(B, S, DPAD)[:, :, :EMBED_DIM]


# speedup vs baseline: 3.5624x; 1.3154x over previous
import functools

import jax
import jax.numpy as jnp
from jax import lax
from jax.experimental import pallas as pl
from jax.experimental.pallas import tpu as pltpu
from jax.experimental.pallas import tpu_sc as plsc

VOCAB = 256
EMBED_DIM = 100
DPAD = 128

NC = 2
NS = 16
NW = NC * NS

G = 128
CHUNK = G
NBUF = 4


def _emb_kernel(n_chunks_per_w):
    n_iter = n_chunks_per_w // NBUF

    def body(idx_hbm, table_hbm, out_hbm, idx_v, rows_v, tab_sh, idx_sem,
             gat_sem, out_sem):
        s = lax.axis_index("s")
        wid = s * NC + lax.axis_index("c")
        base = wid * n_chunks_per_w
        bias = s * VOCAB

        @pl.when(s == 0)
        def _():
            pltpu.sync_copy(table_hbm, tab_sh)
        plsc.subcore_barrier()

        def drain_idx(b):
            pltpu.make_async_copy(idx_hbm.at[0], idx_v.at[b],
                                  idx_sem.at[b]).wait()

        def drain_gat(b):
            pltpu.make_async_copy(out_hbm.at[pl.ds(0, CHUNK)], rows_v.at[b],
                                  gat_sem.at[b]).wait()

        def drain_out(b):
            pltpu.make_async_copy(rows_v.at[b], out_hbm.at[pl.ds(0, CHUNK)],
                                  out_sem.at[b]).wait()

        def bias_idx(b):
            for k in range(G // 16):
                idx_v.at[b][0, pl.ds(k * 16, 16)] = \
                    idx_v.at[b][0, pl.ds(k * 16, 16)] + bias

        def fetch_idx(g, b):
            pltpu.async_copy(idx_hbm.at[base + g], idx_v.at[b],
                             idx_sem.at[b])

        def fire_gather(b):
            pltpu.async_copy(tab_sh.at[idx_v.at[b].at[0]], rows_v.at[b],
                             gat_sem.at[b])

        def fire_writeback(g, b):
            pltpu.async_copy(rows_v.at[b],
                             out_hbm.at[pl.ds((base + g) * CHUNK, CHUNK)],
                             out_sem.at[b])

        for b in range(NBUF):
            fetch_idx(b, b)

        def loop_body(t, carry):
            for b in range(NBUF):
                g = t * NBUF + b
                drain_idx(b)

                @pl.when(t > 0)
                def _():
                    drain_out(b)

                bias_idx(b)
                fire_gather(b)

                b2 = (b - 2) % NBUF

                def complete(gp):
                    drain_gat(b2)
                    fire_writeback(gp, b2)

                if b >= 2:
                    complete(g - 2)

                    @pl.when(t < n_iter - 1)
                    def _():
                        fetch_idx(g + 2, b2)
                else:
                    @pl.when(t > 0)
                    def _():
                        complete(g - 2)
                        fetch_idx(g + 2, b2)
            return carry

        lax.fori_loop(0, n_iter, loop_body, 0)
        last = n_chunks_per_w
        for gp in (last - 2, last - 1):
            b2 = gp % NBUF
            drain_gat(b2)
            fire_writeback(gp, b2)
        for b in range(NBUF):
            drain_out(b)

    return body


def kernel(x, table):
    B, S = x.shape
    n = B * S
    assert n % (NW * CHUNK * NBUF) == 0
    n_chunks_per_w = n // (NW * CHUNK)

    idx = x.reshape(n // CHUNK, 1, G).astype(jnp.int32)
    table_p = jnp.tile(jnp.pad(table, ((0, 0), (0, DPAD - EMBED_DIM))),
                       (NS, 1))

    mesh = plsc.VectorSubcoreMesh(core_axis_name="c", subcore_axis_name="s")
    run = functools.partial(
        pl.kernel,
        mesh=mesh,
        out_type=jax.ShapeDtypeStruct((n, DPAD), jnp.float32),
        scratch_types=[
            pltpu.VMEM((NBUF, 1, G), jnp.int32),
            pltpu.VMEM((NBUF, CHUNK, DPAD), jnp.float32),
            pltpu.VMEM_SHARED((NS * VOCAB, DPAD), jnp.float32),
            pltpu.SemaphoreType.DMA((NBUF,)),
            pltpu.SemaphoreType.DMA((NBUF,)),
            pltpu.SemaphoreType.DMA((NBUF,)),
        ],
        compiler_params=pltpu.CompilerParams(use_tc_tiling_on_sc=False),
    )(_emb_kernel(n_chunks_per_w))

    out = run(idx, table_p)
    return out.reshape(B, S, DPAD)[:, :, :EMBED_DIM]
